# baseline (speedup 1.0000x reference)
import functools

import jax
import jax.numpy as jnp
from jax import lax
from jax.experimental import pallas as pl
from jax.experimental.pallas import tpu as pltpu
from jax.experimental.pallas import tpu_sc as plsc

BATCH = 16384
LATENT = 128
NC = 2
NS = 16
L = 16
NW = NC * NS
BPW = BATCH // NW
CHUNK = 128
NCHUNK = BPW // CHUNK
GROUPS = CHUNK // L
UNROLL = 8


def _mf_body(uid_hbm, iid_hbm, avg_hbm, p_hbm, q_hbm, ub_hbm, ib_hbm,
             out_hbm, uidx, iidx, p_rows, q_rows, ub_v, ib_v, avg_v, out_v,
             sem_p, sem_q, sem_b):
    wid = lax.axis_index("s") * NC + lax.axis_index("c")
    base = wid * BPW

    for k in range(NCHUNK):
        cb = base + k * CHUNK
        pltpu.sync_copy(uid_hbm.at[pl.ds(cb, CHUNK)], uidx)
        pltpu.sync_copy(iid_hbm.at[pl.ds(cb, CHUNK)], iidx)
        cp_p = pltpu.async_copy(p_hbm.at[uidx], p_rows, sem_p)
        cp_q = pltpu.async_copy(q_hbm.at[iidx], q_rows, sem_q)
        cp_ub = pltpu.async_copy(ub_hbm.at[uidx], ub_v, sem_b)
        cp_ib = pltpu.async_copy(ib_hbm.at[iidx], ib_v, sem_b)
        pltpu.sync_copy(avg_hbm.at[pl.ds(cb, CHUNK)], avg_v)
        cp_p.wait()
        cp_q.wait()
        cp_ub.wait()
        cp_ib.wait()

        for g in range(GROUPS):
            row0 = g * L
            rows = lax.iota(jnp.int32, L) + row0

            def dbody(j, acc, rows=rows):
                d0 = j * UNROLL
                for u in range(UNROLL):
                    col = jnp.full((L,), d0 + u, jnp.int32)
                    pv = plsc.load_gather(p_rows, [rows, col])
                    qv = plsc.load_gather(q_rows, [rows, col])
                    acc = acc + pv * qv
                return acc

            acc = lax.fori_loop(0, LATENT // UNROLL, dbody,
                                jnp.zeros((L,), jnp.float32))
            tot = (acc + ub_v[pl.ds(row0, L)] + ib_v[pl.ds(row0, L)]
                   + avg_v[pl.ds(row0, L)])
            out_v[pl.ds(row0, L)] = tot

        pltpu.sync_copy(out_v, out_hbm.at[pl.ds(cb, CHUNK)])


@jax.jit
def _mf(user_id, item_id, avg, P, Q, ub, ib):
    mesh = plsc.VectorSubcoreMesh(core_axis_name="c", subcore_axis_name="s")
    return pl.kernel(
        _mf_body,
        mesh=mesh,
        compiler_params=pltpu.CompilerParams(needs_layout_passes=False),
        out_type=jax.ShapeDtypeStruct((BATCH,), jnp.float32),
        scratch_types=[
            pltpu.VMEM((CHUNK,), jnp.int32),
            pltpu.VMEM((CHUNK,), jnp.int32),
            pltpu.VMEM((CHUNK, LATENT), jnp.float32),
            pltpu.VMEM((CHUNK, LATENT), jnp.float32),
            pltpu.VMEM((CHUNK,), jnp.float32),
            pltpu.VMEM((CHUNK,), jnp.float32),
            pltpu.VMEM((CHUNK,), jnp.float32),
            pltpu.VMEM((CHUNK,), jnp.float32),
            pltpu.SemaphoreType.DMA,
            pltpu.SemaphoreType.DMA,
            pltpu.SemaphoreType.DMA,
        ],
    )(user_id, item_id, avg, P, Q, ub, ib)


def kernel(user_id, item_id, avg_score, P, Q, user_bias, item_bias):
    out = _mf(user_id.astype(jnp.int32), item_id.astype(jnp.int32),
              avg_score.reshape(-1), P, Q,
              user_bias.reshape(-1), item_bias.reshape(-1))
    return out.reshape(BATCH, 1)


# speedup vs baseline: 2.2990x; 2.2990x over previous
import jax
import jax.numpy as jnp
from jax import lax
from jax.experimental import pallas as pl
from jax.experimental.pallas import tpu as pltpu
from jax.experimental.pallas import tpu_sc as plsc

BATCH = 16384
LATENT = 128
NC = 2
NS = 16
L = 16
NW = NC * NS
BPW = BATCH // NW
CHUNK = 128
NCHUNK = BPW // CHUNK
GROUPS = CHUNK // L
UNROLL = 16


def _mf_body(uid_hbm, iid_hbm, avg_hbm, p_hbm, q_hbm, ub_hbm, ib_hbm,
             out_hbm, uidx, iidx, p0, p1, q0, q1, ub_v, ib_v, avg_v, out_v,
             *sems):
    wid = lax.axis_index("s") * NC + lax.axis_index("c")
    base = wid * BPW
    p_bufs = (p0, p1)
    q_bufs = (q0, q1)
    sem_p = sems[0:2]
    sem_q = sems[2:4]
    sem_ub = sems[4:6]
    sem_ib = sems[6:8]

    for k in range(NCHUNK):
        cb = base + k * CHUNK
        pltpu.sync_copy(uid_hbm.at[pl.ds(cb, CHUNK)], uidx.at[k])
        pltpu.sync_copy(iid_hbm.at[pl.ds(cb, CHUNK)], iidx.at[k])
        pltpu.sync_copy(avg_hbm.at[pl.ds(cb, CHUNK)], avg_v.at[k])

    def issue(k):
        buf = k % 2
        return (pltpu.async_copy(p_hbm.at[uidx.at[k]], p_bufs[buf], sem_p[buf]),
                pltpu.async_copy(q_hbm.at[iidx.at[k]], q_bufs[buf], sem_q[buf]),
                pltpu.async_copy(ub_hbm.at[uidx.at[k]], ub_v.at[k], sem_ub[buf]),
                pltpu.async_copy(ib_hbm.at[iidx.at[k]], ib_v.at[k], sem_ib[buf]))

    lane = lax.iota(jnp.int32, L)

    inflight = issue(0)
    for k in range(NCHUNK):
        for cp in inflight:
            cp.wait()
        if k + 1 < NCHUNK:
            inflight = issue(k + 1)
        p_rows, q_rows = p_bufs[k % 2], q_bufs[k % 2]

        for g in range(GROUPS):
            rows16 = lane + (g * L)

            def dbody(m, acc, rows16=rows16, p_rows=p_rows, q_rows=q_rows):
                c0 = m * UNROLL
                for u in range(UNROLL):
                    col = (lane + (c0 + u)) & (LATENT - 1)
                    pv = plsc.load_gather(p_rows, [rows16, col])
                    qv = plsc.load_gather(q_rows, [rows16, col])
                    acc = acc + pv * qv
                return acc

            acc = lax.fori_loop(0, LATENT // UNROLL, dbody,
                                jnp.zeros((L,), jnp.float32))
            tot = (acc + ub_v[k, pl.ds(g * L, L)] + ib_v[k, pl.ds(g * L, L)]
                   + avg_v[k, pl.ds(g * L, L)])
            out_v[pl.ds(g * L, L)] = tot

        pltpu.sync_copy(out_v, out_hbm.at[pl.ds(base + k * CHUNK, CHUNK)])


@jax.jit
def _mf(user_id, item_id, avg, P, Q, ub, ib):
    mesh = plsc.VectorSubcoreMesh(core_axis_name="c", subcore_axis_name="s")
    return pl.kernel(
        _mf_body,
        mesh=mesh,
        compiler_params=pltpu.CompilerParams(needs_layout_passes=False),
        out_type=jax.ShapeDtypeStruct((BATCH,), jnp.float32),
        scratch_types=[
            pltpu.VMEM((NCHUNK, CHUNK), jnp.int32),
            pltpu.VMEM((NCHUNK, CHUNK), jnp.int32),
            pltpu.VMEM((CHUNK, LATENT), jnp.float32),
            pltpu.VMEM((CHUNK, LATENT), jnp.float32),
            pltpu.VMEM((CHUNK, LATENT), jnp.float32),
            pltpu.VMEM((CHUNK, LATENT), jnp.float32),
            pltpu.VMEM((NCHUNK, CHUNK), jnp.float32),
            pltpu.VMEM((NCHUNK, CHUNK), jnp.float32),
            pltpu.VMEM((NCHUNK, CHUNK), jnp.float32),
            pltpu.VMEM((CHUNK,), jnp.float32),
        ] + [pltpu.SemaphoreType.DMA] * 8,
    )(user_id, item_id, avg, P, Q, ub, ib)


def kernel(user_id, item_id, avg_score, P, Q, user_bias, item_bias):
    out = _mf(user_id.astype(jnp.int32), item_id.astype(jnp.int32),
              avg_score.reshape(-1), P, Q,
              user_bias.reshape(-1), item_bias.reshape(-1))
    return out.reshape(BATCH, 1)


# speedup vs baseline: 2.8374x; 1.2342x over previous
import jax
import jax.numpy as jnp
from jax import lax
from jax.experimental import pallas as pl
from jax.experimental.pallas import tpu as pltpu
from jax.experimental.pallas import tpu_sc as plsc

BATCH = 16384
LATENT = 128
NC = 2
NS = 16
L = 16
NW = NC * NS
BPW = BATCH // NW
CHUNK = 128
NCHUNK = BPW // CHUNK
GROUPS = CHUNK // L
UNROLL = 16


def _mf_body(uid_hbm, iid_hbm, avg_hbm, p_hbm, q_hbm, ub_hbm, ib_hbm,
             out_hbm, uidx, iidx, p0, p1, q0, q1, ub_v, ib_v, avg_v, out_v,
             *sems):
    wid = lax.axis_index("s") * NC + lax.axis_index("c")
    base = wid * BPW
    p_bufs = (p0, p1)
    q_bufs = (q0, q1)
    sem_p = sems[0:2]
    sem_q = sems[2:4]
    sem_ub = sems[4:6]
    sem_ib = sems[6:8]
    sem_ids = sems[8]
    sem_out = sems[9]

    cp_uid = pltpu.async_copy(uid_hbm.at[pl.ds(base, BPW)], uidx, sem_ids)
    cp_iid = pltpu.async_copy(iid_hbm.at[pl.ds(base, BPW)], iidx, sem_ids)
    cp_avg = pltpu.async_copy(avg_hbm.at[pl.ds(base, BPW)], avg_v, sem_ids)
    cp_uid.wait()
    cp_iid.wait()

    def issue(k):
        buf = k % 2
        uk = uidx.at[pl.ds(k * CHUNK, CHUNK)]
        ik = iidx.at[pl.ds(k * CHUNK, CHUNK)]
        return (pltpu.async_copy(p_hbm.at[uk], p_bufs[buf], sem_p[buf]),
                pltpu.async_copy(q_hbm.at[ik], q_bufs[buf], sem_q[buf]),
                pltpu.async_copy(ub_hbm.at[uk], ub_v.at[k], sem_ub[buf]),
                pltpu.async_copy(ib_hbm.at[ik], ib_v.at[k], sem_ib[buf]))

    lane = lax.iota(jnp.int32, L)
    out_cps = []

    inflight = issue(0)
    cp_avg.wait()
    for k in range(NCHUNK):
        if k + 1 < NCHUNK:
            nxt = issue(k + 1)
        for cp in inflight:
            cp.wait()
        if k + 1 < NCHUNK:
            inflight = nxt
        p_rows, q_rows = p_bufs[k % 2], q_bufs[k % 2]

        for g in range(GROUPS):
            rows16 = lane + (g * L)

            def dbody(m, accs, rows16=rows16, p_rows=p_rows, q_rows=q_rows):
                a0, a1 = accs
                c0 = m * UNROLL
                for u in range(UNROLL):
                    col = (lane + (c0 + u)) & (LATENT - 1)
                    pv = plsc.load_gather(p_rows, [rows16, col])
                    qv = plsc.load_gather(q_rows, [rows16, col])
                    if u % 2 == 0:
                        a0 = a0 + pv * qv
                    else:
                        a1 = a1 + pv * qv
                return a0, a1

            zero = jnp.zeros((L,), jnp.float32)
            a0, a1 = lax.fori_loop(0, LATENT // UNROLL, dbody, (zero, zero))
            tot = ((a0 + a1)
                   + ub_v[k, pl.ds(g * L, L)] + ib_v[k, pl.ds(g * L, L)]
                   + avg_v[pl.ds(k * CHUNK + g * L, L)])
            out_v[k, pl.ds(g * L, L)] = tot

        out_cps.append(pltpu.async_copy(
            out_v.at[k], out_hbm.at[pl.ds(base + k * CHUNK, CHUNK)], sem_out))

    for cp in out_cps:
        cp.wait()


@jax.jit
def _mf(user_id, item_id, avg, P, Q, ub, ib):
    mesh = plsc.VectorSubcoreMesh(core_axis_name="c", subcore_axis_name="s")
    return pl.kernel(
        _mf_body,
        mesh=mesh,
        compiler_params=pltpu.CompilerParams(needs_layout_passes=False),
        out_type=jax.ShapeDtypeStruct((BATCH,), jnp.float32),
        scratch_types=[
            pltpu.VMEM((BPW,), jnp.int32),
            pltpu.VMEM((BPW,), jnp.int32),
            pltpu.VMEM((CHUNK, LATENT), jnp.float32),
            pltpu.VMEM((CHUNK, LATENT), jnp.float32),
            pltpu.VMEM((CHUNK, LATENT), jnp.float32),
            pltpu.VMEM((CHUNK, LATENT), jnp.float32),
            pltpu.VMEM((NCHUNK, CHUNK), jnp.float32),
            pltpu.VMEM((NCHUNK, CHUNK), jnp.float32),
            pltpu.VMEM((BPW,), jnp.float32),
            pltpu.VMEM((NCHUNK, CHUNK), jnp.float32),
        ] + [pltpu.SemaphoreType.DMA] * 10,
    )(user_id, item_id, avg, P, Q, ub, ib)


def kernel(user_id, item_id, avg_score, P, Q, user_bias, item_bias):
    out = _mf(user_id.astype(jnp.int32), item_id.astype(jnp.int32),
              avg_score.reshape(-1), P, Q,
              user_bias.reshape(-1), item_bias.reshape(-1))
    return out.reshape(BATCH, 1)
